# Initial kernel scaffold; baseline (speedup 1.0000x reference)
import jax
import jax.numpy as jnp
from jax.experimental import pallas as pl


def kernel(x, w_gate, w1, b1, w2, b2, w3, b3):
    raise NotImplementedError("write your pallas kernel here")


# speedup vs baseline: 1.2281x; 1.2281x over previous
import functools

import jax
import jax.numpy as jnp
from jax.experimental import pallas as pl
from jax.experimental.pallas import tpu as pltpu

B, C, L = 64, 384, 196
E, K = 8, 2
BOT = 96
LOSS_COEF = 0.01


def _gating_kernel(x_ref, wg_ref, idx_ref, gv_ref, loss_ref):
    x = x_ref[...]
    pooled = jnp.mean(x, axis=-1)
    logits = jnp.dot(pooled, wg_ref[...],
                     preferred_element_type=jnp.float32)
    mx = jnp.max(logits, axis=1, keepdims=True)
    ex = jnp.exp(logits - mx)
    probs = ex / jnp.sum(ex, axis=1, keepdims=True)

    iota = jax.lax.broadcasted_iota(jnp.int32, (B, E), 1)
    m1 = jnp.max(probs, axis=1, keepdims=True)
    a1 = jnp.min(jnp.where(probs == m1, iota, E), axis=1)
    masked = jnp.where(iota == a1[:, None], -1.0, probs)
    m2 = jnp.max(masked, axis=1, keepdims=True)
    a2 = jnp.min(jnp.where(masked == m2, iota, E), axis=1)

    e2 = jnp.exp(m2 - m1)
    g1 = 1.0 / (1.0 + e2)
    g2 = e2 * g1

    oh1 = (iota == a1[:, None]).astype(jnp.float32)
    oh2 = (iota == a2[:, None]).astype(jnp.float32)
    importance = jnp.sum(oh1 * g1 + oh2 * g2, axis=0)
    load = jnp.sum(oh1 + oh2, axis=0)

    def cv2(v):
        mean = jnp.mean(v)
        var = jnp.sum((v - mean) ** 2) / (E - 1)
        return var / (mean * mean + 1e-10)

    loss = LOSS_COEF * (cv2(importance) + cv2(load))
    loss_ref[...] = jnp.reshape(loss, (1, 1))
    idx_ref[...] = jnp.stack([a1, a2], axis=1).astype(jnp.int32)
    gv_ref[...] = jnp.concatenate([g1, g2], axis=1)


def _dispatch_kernel(idx_ref, gv_ref, x_ref, w1_ref, w2_ref, w3_ref,
                     b1_ref, b2_ref, b3_ref, y_ref):
    b = pl.program_id(0)
    k = pl.program_id(1)
    e = idx_ref[b, k]
    g = gv_ref[b, k]

    xb = x_ref[0]
    h = jnp.dot(w1_ref[0], xb, preferred_element_type=jnp.float32)
    h = jnp.maximum(h + b1_ref[e][:, None], 0.0)

    zero = jnp.zeros((BOT, 1), dtype=jnp.float32)
    hp = jnp.concatenate([zero, h, zero], axis=1)
    acc = jnp.dot(w2_ref[0, 0], hp[:, 0:L], preferred_element_type=jnp.float32)
    acc += jnp.dot(w2_ref[0, 1], hp[:, 1:L + 1], preferred_element_type=jnp.float32)
    acc += jnp.dot(w2_ref[0, 2], hp[:, 2:L + 2], preferred_element_type=jnp.float32)
    h2 = jnp.maximum(acc + b2_ref[e][:, None], 0.0)

    y = jnp.dot(w3_ref[0], h2, preferred_element_type=jnp.float32)
    y = y + b3_ref[e][:, None]
    out = g * jnp.maximum(y + xb, 0.0)

    @pl.when(k == 0)
    def _():
        y_ref[0] = out

    @pl.when(k != 0)
    def _():
        y_ref[0] += out


@jax.jit
def kernel(x, w_gate, w1, b1, w2, b2, w3, b3):
    idx, gv, loss = pl.pallas_call(
        _gating_kernel,
        out_shape=(
            jax.ShapeDtypeStruct((B, K), jnp.int32),
            jax.ShapeDtypeStruct((B, K), jnp.float32),
            jax.ShapeDtypeStruct((1, 1), jnp.float32),
        ),
    )(x, w_gate)

    w1r = w1.reshape(E, BOT, C)
    w2r = jnp.transpose(w2, (0, 3, 1, 2))
    w3r = w3.reshape(E, C, BOT)

    grid_spec = pltpu.PrefetchScalarGridSpec(
        num_scalar_prefetch=2,
        grid=(B, K),
        in_specs=[
            pl.BlockSpec((1, C, L), lambda b, k, idx, gv: (b, 0, 0)),
            pl.BlockSpec((1, BOT, C), lambda b, k, idx, gv: (idx[b, k], 0, 0)),
            pl.BlockSpec((1, 3, BOT, BOT), lambda b, k, idx, gv: (idx[b, k], 0, 0, 0)),
            pl.BlockSpec((1, C, BOT), lambda b, k, idx, gv: (idx[b, k], 0, 0)),
            pl.BlockSpec((E, BOT), lambda b, k, idx, gv: (0, 0)),
            pl.BlockSpec((E, BOT), lambda b, k, idx, gv: (0, 0)),
            pl.BlockSpec((E, C), lambda b, k, idx, gv: (0, 0)),
        ],
        out_specs=pl.BlockSpec((1, C, L), lambda b, k, idx, gv: (b, 0, 0)),
    )
    y = pl.pallas_call(
        _dispatch_kernel,
        grid_spec=grid_spec,
        out_shape=jax.ShapeDtypeStruct((B, C, L), jnp.float32),
    )(idx, gv, x, w1r, w2r, w3r, b1, b2, b3)

    return (y, loss.reshape(()))


# speedup vs baseline: 1.2554x; 1.0223x over previous
import functools

import jax
import jax.numpy as jnp
from jax.experimental import pallas as pl
from jax.experimental.pallas import tpu as pltpu

B, C, L = 64, 384, 196
E, K = 8, 2
BOT = 96
LOSS_COEF = 0.01


def _gating_kernel(x_ref, wg_ref, idx_ref, gv_ref, loss_ref):
    x = x_ref[...]
    pooled = jnp.mean(x, axis=-1)
    logits = jnp.dot(pooled, wg_ref[...],
                     preferred_element_type=jnp.float32)
    mx = jnp.max(logits, axis=1, keepdims=True)
    ex = jnp.exp(logits - mx)
    probs = ex / jnp.sum(ex, axis=1, keepdims=True)

    iota = jax.lax.broadcasted_iota(jnp.int32, (B, E), 1)
    m1 = jnp.max(probs, axis=1, keepdims=True)
    a1 = jnp.min(jnp.where(probs == m1, iota, E), axis=1)
    masked = jnp.where(iota == a1[:, None], -1.0, probs)
    m2 = jnp.max(masked, axis=1, keepdims=True)
    a2 = jnp.min(jnp.where(masked == m2, iota, E), axis=1)

    e2 = jnp.exp(m2 - m1)
    g1 = 1.0 / (1.0 + e2)
    g2 = e2 * g1

    oh1 = (iota == a1[:, None]).astype(jnp.float32)
    oh2 = (iota == a2[:, None]).astype(jnp.float32)
    importance = jnp.sum(oh1 * g1 + oh2 * g2, axis=0)
    load = jnp.sum(oh1 + oh2, axis=0)

    def cv2(v):
        mean = jnp.mean(v)
        var = jnp.sum((v - mean) ** 2) / (E - 1)
        return var / (mean * mean + 1e-10)

    loss = LOSS_COEF * (cv2(importance) + cv2(load))
    loss_ref[...] = jnp.reshape(loss, (1, 1))
    idx_ref[...] = jnp.stack([a1, a2], axis=1).astype(jnp.int32)
    gv_ref[...] = jnp.concatenate([g1, g2], axis=1)


def _dispatch_kernel(idx_ref, gv_ref, x_ref, w1_ref, w2_ref, w3_ref,
                     b1_ref, b2_ref, b3_ref, y_ref):
    b = pl.program_id(0)
    k = pl.program_id(1)
    e = idx_ref[b, k]
    g = gv_ref[b, k]

    xb = x_ref[0]
    xb_bf = xb.astype(jnp.bfloat16)
    h = jnp.dot(w1_ref[0], xb_bf, preferred_element_type=jnp.float32)
    h = jnp.maximum(h + b1_ref[e][:, None], 0.0)
    h = h.astype(jnp.bfloat16)

    zero = jnp.zeros((BOT, 1), dtype=jnp.bfloat16)
    hp = jnp.concatenate([zero, h, zero], axis=1)
    acc = jnp.dot(w2_ref[0, 0], hp[:, 0:L], preferred_element_type=jnp.float32)
    acc += jnp.dot(w2_ref[0, 1], hp[:, 1:L + 1], preferred_element_type=jnp.float32)
    acc += jnp.dot(w2_ref[0, 2], hp[:, 2:L + 2], preferred_element_type=jnp.float32)
    h2 = jnp.maximum(acc + b2_ref[e][:, None], 0.0)
    h2 = h2.astype(jnp.bfloat16)

    y = jnp.dot(w3_ref[0], h2, preferred_element_type=jnp.float32)
    y = y + b3_ref[e][:, None]
    out = g * jnp.maximum(y + xb, 0.0)

    @pl.when(k == 0)
    def _():
        y_ref[0] = out

    @pl.when(k != 0)
    def _():
        y_ref[0] += out


@jax.jit
def kernel(x, w_gate, w1, b1, w2, b2, w3, b3):
    idx, gv, loss = pl.pallas_call(
        _gating_kernel,
        out_shape=(
            jax.ShapeDtypeStruct((B, K), jnp.int32),
            jax.ShapeDtypeStruct((B, K), jnp.float32),
            jax.ShapeDtypeStruct((1, 1), jnp.float32),
        ),
    )(x, w_gate)

    w1r = w1.reshape(E, BOT, C).astype(jnp.bfloat16)
    w2r = jnp.transpose(w2, (0, 3, 1, 2)).astype(jnp.bfloat16)
    w3r = w3.reshape(E, C, BOT).astype(jnp.bfloat16)

    grid_spec = pltpu.PrefetchScalarGridSpec(
        num_scalar_prefetch=2,
        grid=(B, K),
        in_specs=[
            pl.BlockSpec((1, C, L), lambda b, k, idx, gv: (b, 0, 0)),
            pl.BlockSpec((1, BOT, C), lambda b, k, idx, gv: (idx[b, k], 0, 0)),
            pl.BlockSpec((1, 3, BOT, BOT), lambda b, k, idx, gv: (idx[b, k], 0, 0, 0)),
            pl.BlockSpec((1, C, BOT), lambda b, k, idx, gv: (idx[b, k], 0, 0)),
            pl.BlockSpec((E, BOT), lambda b, k, idx, gv: (0, 0)),
            pl.BlockSpec((E, BOT), lambda b, k, idx, gv: (0, 0)),
            pl.BlockSpec((E, C), lambda b, k, idx, gv: (0, 0)),
        ],
        out_specs=pl.BlockSpec((1, C, L), lambda b, k, idx, gv: (b, 0, 0)),
    )
    y = pl.pallas_call(
        _dispatch_kernel,
        grid_spec=grid_spec,
        out_shape=jax.ShapeDtypeStruct((B, C, L), jnp.float32),
    )(idx, gv, x, w1r, w2r, w3r, b1, b2, b3)

    return (y, loss.reshape(()))


# speedup vs baseline: 1.5975x; 1.2724x over previous
import functools

import jax
import jax.numpy as jnp
from jax.experimental import pallas as pl
from jax.experimental.pallas import tpu as pltpu

B, C, L = 64, 384, 196
E, K = 8, 2
BOT = 96
LOSS_COEF = 0.01
G = 2


def _gating_kernel(x_ref, wg_ref, idx_ref, gv_ref, loss_ref):
    x = x_ref[...]
    pooled = jnp.mean(x, axis=-1)
    logits = jnp.dot(pooled, wg_ref[...],
                     preferred_element_type=jnp.float32)
    mx = jnp.max(logits, axis=1, keepdims=True)
    ex = jnp.exp(logits - mx)
    probs = ex / jnp.sum(ex, axis=1, keepdims=True)

    iota = jax.lax.broadcasted_iota(jnp.int32, (B, E), 1)
    m1 = jnp.max(probs, axis=1, keepdims=True)
    a1 = jnp.min(jnp.where(probs == m1, iota, E), axis=1)
    masked = jnp.where(iota == a1[:, None], -1.0, probs)
    m2 = jnp.max(masked, axis=1, keepdims=True)
    a2 = jnp.min(jnp.where(masked == m2, iota, E), axis=1)

    e2 = jnp.exp(m2 - m1)
    g1 = 1.0 / (1.0 + e2)
    g2 = e2 * g1

    oh1 = (iota == a1[:, None]).astype(jnp.float32)
    oh2 = (iota == a2[:, None]).astype(jnp.float32)
    importance = jnp.sum(oh1 * g1 + oh2 * g2, axis=0)
    load = jnp.sum(oh1 + oh2, axis=0)

    def cv2(v):
        mean = jnp.mean(v)
        var = jnp.sum((v - mean) ** 2) / (E - 1)
        return var / (mean * mean + 1e-10)

    loss = LOSS_COEF * (cv2(importance) + cv2(load))
    loss_ref[...] = jnp.reshape(loss, (1, 1))
    idx_ref[...] = jnp.stack([a1, a2], axis=1).astype(jnp.int32)
    gv_ref[...] = jnp.concatenate([g1, g2], axis=1)


def _expert_chain(xb_bf, w1, w2, w3, b1v, b2v, b3v):
    h = jnp.dot(w1, xb_bf, preferred_element_type=jnp.float32)
    h = jnp.maximum(h + b1v[:, None], 0.0).astype(jnp.bfloat16)
    zero = jnp.zeros((BOT, 1), dtype=jnp.bfloat16)
    hp = jnp.concatenate([zero, h, zero], axis=1)
    acc = jnp.dot(w2[0], hp[:, 0:L], preferred_element_type=jnp.float32)
    acc += jnp.dot(w2[1], hp[:, 1:L + 1], preferred_element_type=jnp.float32)
    acc += jnp.dot(w2[2], hp[:, 2:L + 2], preferred_element_type=jnp.float32)
    h2 = jnp.maximum(acc + b2v[:, None], 0.0).astype(jnp.bfloat16)
    y = jnp.dot(w3, h2, preferred_element_type=jnp.float32)
    return y + b3v[:, None]


def _dispatch_kernel(idx_ref, gv_ref, x_ref,
                     w1_00, w2_00, w3_00, w1_01, w2_01, w3_01,
                     w1_10, w2_10, w3_10, w1_11, w2_11, w3_11,
                     b1_ref, b2_ref, b3_ref, y_ref):
    i = pl.program_id(0)
    wsets = ((w1_00, w2_00, w3_00), (w1_01, w2_01, w3_01),
             (w1_10, w2_10, w3_10), (w1_11, w2_11, w3_11))
    for di in range(G):
        xb = x_ref[di]
        xb_bf = xb.astype(jnp.bfloat16)
        acc = None
        for k in range(K):
            b = i * G + di
            e = idx_ref[b, k]
            g = gv_ref[b, k]
            w1r, w2r, w3r = wsets[di * K + k]
            y = _expert_chain(xb_bf, w1r[0], w2r[0], w3r[0],
                              b1_ref[e], b2_ref[e], b3_ref[e])
            term = g * jnp.maximum(y + xb, 0.0)
            acc = term if acc is None else acc + term
        y_ref[di] = acc


@jax.jit
def kernel(x, w_gate, w1, b1, w2, b2, w3, b3):
    idx, gv, loss = pl.pallas_call(
        _gating_kernel,
        out_shape=(
            jax.ShapeDtypeStruct((B, K), jnp.int32),
            jax.ShapeDtypeStruct((B, K), jnp.float32),
            jax.ShapeDtypeStruct((1, 1), jnp.float32),
        ),
    )(x, w_gate)

    w1r = w1.reshape(E, BOT, C).astype(jnp.bfloat16)
    w2r = jnp.transpose(w2, (0, 3, 1, 2)).astype(jnp.bfloat16)
    w3r = w3.reshape(E, C, BOT).astype(jnp.bfloat16)

    def wspecs(di, k):
        return [
            pl.BlockSpec((1, BOT, C), lambda i, idx, gv, di=di, k=k: (idx[i * G + di, k], 0, 0)),
            pl.BlockSpec((1, 3, BOT, BOT), lambda i, idx, gv, di=di, k=k: (idx[i * G + di, k], 0, 0, 0)),
            pl.BlockSpec((1, C, BOT), lambda i, idx, gv, di=di, k=k: (idx[i * G + di, k], 0, 0)),
        ]

    grid_spec = pltpu.PrefetchScalarGridSpec(
        num_scalar_prefetch=2,
        grid=(B // G,),
        in_specs=[
            pl.BlockSpec((G, C, L), lambda i, idx, gv: (i, 0, 0)),
            *wspecs(0, 0), *wspecs(0, 1), *wspecs(1, 0), *wspecs(1, 1),
            pl.BlockSpec((E, BOT), lambda i, idx, gv: (0, 0)),
            pl.BlockSpec((E, BOT), lambda i, idx, gv: (0, 0)),
            pl.BlockSpec((E, C), lambda i, idx, gv: (0, 0)),
        ],
        out_specs=pl.BlockSpec((G, C, L), lambda i, idx, gv: (i, 0, 0)),
    )
    y = pl.pallas_call(
        _dispatch_kernel,
        grid_spec=grid_spec,
        out_shape=jax.ShapeDtypeStruct((B, C, L), jnp.float32),
    )(idx, gv, x, w1r, w2r, w3r, w1r, w2r, w3r, w1r, w2r, w3r, w1r, w2r, w3r,
      b1, b2, b3)

    return (y, loss.reshape(()))


# speedup vs baseline: 1.6922x; 1.0593x over previous
import jax
import jax.numpy as jnp
from jax.experimental import pallas as pl
from jax.experimental.pallas import tpu as pltpu

B, C, L = 64, 384, 196
E, K = 8, 2
BOT = 96
LOSS_COEF = 0.01
G = 4
NCH = 8
BCH = B // NCH


def _gating_kernel(x_ref, wg_ref, w1_ref, w2f_ref, w3_ref,
                   idx_ref, gv_ref, loss_ref, w1b_ref, w2b_ref, w3b_ref,
                   pooled_ref):
    i = pl.program_id(0)

    @pl.when(i == 0)
    def _():
        w1b_ref[...] = w1_ref[...].astype(jnp.bfloat16)
        w3b_ref[...] = w3_ref[...].astype(jnp.bfloat16)
        rows = jax.lax.broadcasted_iota(jnp.int32, (3 * BOT, BOT), 0)
        cols = jax.lax.broadcasted_iota(jnp.int32, (3 * BOT, BOT), 1)
        for t in range(3):
            sel = (rows == 3 * cols + t).astype(jnp.bfloat16)
            for e in range(E):
                w2b_ref[e, t] = jnp.dot(
                    w2f_ref[e].astype(jnp.bfloat16), sel,
                    preferred_element_type=jnp.float32).astype(jnp.bfloat16)

    pooled_ref[pl.ds(i * BCH, BCH), :] = jnp.mean(x_ref[...], axis=-1)

    @pl.when(i == NCH - 1)
    def _():
        pooled = pooled_ref[...]
        logits = jnp.dot(pooled, wg_ref[...],
                         preferred_element_type=jnp.float32)
        mx = jnp.max(logits, axis=1, keepdims=True)
        ex = jnp.exp(logits - mx)
        probs = ex / jnp.sum(ex, axis=1, keepdims=True)

        iota = jax.lax.broadcasted_iota(jnp.int32, (B, E), 1)
        m1 = jnp.max(probs, axis=1, keepdims=True)
        a1 = jnp.min(jnp.where(probs == m1, iota, E), axis=1)
        masked = jnp.where(iota == a1[:, None], -1.0, probs)
        m2 = jnp.max(masked, axis=1, keepdims=True)
        a2 = jnp.min(jnp.where(masked == m2, iota, E), axis=1)

        e2 = jnp.exp(m2 - m1)
        g1 = 1.0 / (1.0 + e2)
        g2 = e2 * g1

        oh1 = (iota == a1[:, None]).astype(jnp.float32)
        oh2 = (iota == a2[:, None]).astype(jnp.float32)
        importance = jnp.sum(oh1 * g1 + oh2 * g2, axis=0)
        load = jnp.sum(oh1 + oh2, axis=0)

        def cv2(v):
            mean = jnp.mean(v)
            var = jnp.sum((v - mean) ** 2) / (E - 1)
            return var / (mean * mean + 1e-10)

        loss = LOSS_COEF * (cv2(importance) + cv2(load))
        loss_ref[...] = jnp.reshape(loss, (1, 1))
        idx_ref[...] = jnp.stack([a1, a2], axis=1).astype(jnp.int32)
        gv_ref[...] = jnp.concatenate([g1, g2], axis=1)


def _expert_chain(xb_bf, w1, w2, w3, b1v, b2v, b3v):
    h = jnp.dot(w1, xb_bf, preferred_element_type=jnp.float32)
    h = jnp.maximum(h + b1v[:, None], 0.0).astype(jnp.bfloat16)
    zero = jnp.zeros((BOT, 1), dtype=jnp.bfloat16)
    hp = jnp.concatenate([zero, h, zero], axis=1)
    acc = jnp.dot(w2[0], hp[:, 0:L], preferred_element_type=jnp.float32)
    acc += jnp.dot(w2[1], hp[:, 1:L + 1], preferred_element_type=jnp.float32)
    acc += jnp.dot(w2[2], hp[:, 2:L + 2], preferred_element_type=jnp.float32)
    h2 = jnp.maximum(acc + b2v[:, None], 0.0).astype(jnp.bfloat16)
    y = jnp.dot(w3, h2, preferred_element_type=jnp.float32)
    return y + b3v[:, None]


def _dispatch_kernel(idx_ref, gv_ref, x_ref, *rest):
    wrefs = rest[:3 * G * K]
    b1_ref, b2_ref, b3_ref, y_ref = rest[3 * G * K:]
    i = pl.program_id(0)
    for di in range(G):
        xb = x_ref[di]
        xb_bf = xb.astype(jnp.bfloat16)
        acc = None
        for k in range(K):
            b = i * G + di
            e = idx_ref[b, k]
            g = gv_ref[b, k]
            w1r, w2r, w3r = wrefs[3 * (di * K + k):3 * (di * K + k) + 3]
            y = _expert_chain(xb_bf, w1r[0], w2r[0], w3r[0],
                              b1_ref[e], b2_ref[e], b3_ref[e])
            term = g * jnp.maximum(y + xb, 0.0)
            acc = term if acc is None else acc + term
        y_ref[di] = acc


@jax.jit
def kernel(x, w_gate, w1, b1, w2, b2, w3, b3):
    w1f = w1.reshape(E, BOT, C)
    w2f = w2.reshape(E, BOT, BOT * 3)
    w3f = w3.reshape(E, C, BOT)

    idx, gv, loss, w1b, w2b, w3b = pl.pallas_call(
        _gating_kernel,
        grid=(NCH,),
        in_specs=[
            pl.BlockSpec((BCH, C, L), lambda i: (i, 0, 0)),
            pl.BlockSpec((C, E), lambda i: (0, 0)),
            pl.BlockSpec((E, BOT, C), lambda i: (0, 0, 0)),
            pl.BlockSpec((E, BOT, BOT * 3), lambda i: (0, 0, 0)),
            pl.BlockSpec((E, C, BOT), lambda i: (0, 0, 0)),
        ],
        out_specs=(
            pl.BlockSpec((B, K), lambda i: (0, 0)),
            pl.BlockSpec((B, K), lambda i: (0, 0)),
            pl.BlockSpec((1, 1), lambda i: (0, 0)),
            pl.BlockSpec((E, BOT, C), lambda i: (0, 0, 0)),
            pl.BlockSpec((E, 3, BOT, BOT), lambda i: (0, 0, 0, 0)),
            pl.BlockSpec((E, C, BOT), lambda i: (0, 0, 0)),
        ),
        out_shape=(
            jax.ShapeDtypeStruct((B, K), jnp.int32),
            jax.ShapeDtypeStruct((B, K), jnp.float32),
            jax.ShapeDtypeStruct((1, 1), jnp.float32),
            jax.ShapeDtypeStruct((E, BOT, C), jnp.bfloat16),
            jax.ShapeDtypeStruct((E, 3, BOT, BOT), jnp.bfloat16),
            jax.ShapeDtypeStruct((E, C, BOT), jnp.bfloat16),
        ),
        scratch_shapes=[pltpu.VMEM((B, C), jnp.float32)],
    )(x, w_gate, w1f, w2f, w3f)

    def wspecs(c):
        return [
            pl.BlockSpec((1, BOT, C), lambda i, idx, gv, c=c: (idx[i * G + c // K, c % K], 0, 0)),
            pl.BlockSpec((1, 3, BOT, BOT), lambda i, idx, gv, c=c: (idx[i * G + c // K, c % K], 0, 0, 0)),
            pl.BlockSpec((1, C, BOT), lambda i, idx, gv, c=c: (idx[i * G + c // K, c % K], 0, 0)),
        ]

    grid_spec = pltpu.PrefetchScalarGridSpec(
        num_scalar_prefetch=2,
        grid=(B // G,),
        in_specs=[
            pl.BlockSpec((G, C, L), lambda i, idx, gv: (i, 0, 0)),
            *[s for c in range(G * K) for s in wspecs(c)],
            pl.BlockSpec((E, BOT), lambda i, idx, gv: (0, 0)),
            pl.BlockSpec((E, BOT), lambda i, idx, gv: (0, 0)),
            pl.BlockSpec((E, C), lambda i, idx, gv: (0, 0)),
        ],
        out_specs=pl.BlockSpec((G, C, L), lambda i, idx, gv: (i, 0, 0)),
    )
    y = pl.pallas_call(
        _dispatch_kernel,
        grid_spec=grid_spec,
        out_shape=jax.ShapeDtypeStruct((B, C, L), jnp.float32),
    )(idx, gv, x, *([w1b, w2b, w3b] * (G * K)), b1, b2, b3)

    return (y, loss.reshape(()))


# speedup vs baseline: 2.3817x; 1.4075x over previous
import jax
import jax.numpy as jnp
from jax.experimental import pallas as pl
from jax.experimental.pallas import tpu as pltpu

B, C, L = 64, 384, 196
E, K = 8, 2
BOT = 96
LOSS_COEF = 0.01
G = 8
LCH = 28

_DN_CONTRACT1 = (((1,), (1,)), ((), ()))


def _gating_kernel(x_ref, wg_ref, w1_ref, w2_ref, w3_ref,
                   idx_ref, gv_ref, loss_ref, w1b_ref, w2b_ref, w3b_ref,
                   pooled_ref):
    i = pl.program_id(0)

    @pl.when(i == 0)
    def _():
        pooled_ref[...] = jnp.zeros_like(pooled_ref)
        w1b_ref[...] = w1_ref[...].astype(jnp.bfloat16)
        w2b_ref[...] = w2_ref[...].astype(jnp.bfloat16)
        w3b_ref[...] = w3_ref[...].astype(jnp.bfloat16)

    pooled_ref[...] += jnp.sum(x_ref[...], axis=0)

    @pl.when(i == pl.num_programs(0) - 1)
    def _():
        pooled = pooled_ref[...] * (1.0 / L)
        logits = jax.lax.dot_general(pooled, wg_ref[...], _DN_CONTRACT1,
                                     preferred_element_type=jnp.float32)
        mx = jnp.max(logits, axis=1, keepdims=True)
        ex = jnp.exp(logits - mx)
        probs = ex / jnp.sum(ex, axis=1, keepdims=True)

        iota = jax.lax.broadcasted_iota(jnp.int32, (B, E), 1)
        m1 = jnp.max(probs, axis=1, keepdims=True)
        a1 = jnp.min(jnp.where(probs == m1, iota, E), axis=1)
        masked = jnp.where(iota == a1[:, None], -1.0, probs)
        m2 = jnp.max(masked, axis=1, keepdims=True)
        a2 = jnp.min(jnp.where(masked == m2, iota, E), axis=1)

        e2 = jnp.exp(m2 - m1)
        g1 = 1.0 / (1.0 + e2)
        g2 = e2 * g1

        oh1 = (iota == a1[:, None]).astype(jnp.float32)
        oh2 = (iota == a2[:, None]).astype(jnp.float32)
        importance = jnp.sum(oh1 * g1 + oh2 * g2, axis=0)
        load = jnp.sum(oh1 + oh2, axis=0)

        def cv2(v):
            mean = jnp.mean(v)
            var = jnp.sum((v - mean) ** 2) / (E - 1)
            return var / (mean * mean + 1e-10)

        loss = LOSS_COEF * (cv2(importance) + cv2(load))
        loss_ref[...] = jnp.reshape(loss, (1, 1))
        idx_ref[...] = jnp.stack([a1, a2], axis=1).astype(jnp.int32)
        gv_ref[...] = jnp.concatenate([g1, g2], axis=1)


def _expert_chain(xb_bf, w1, w2, w3, b1v, b2v, b3v):
    h = jax.lax.dot_general(xb_bf, w1, _DN_CONTRACT1,
                            preferred_element_type=jnp.float32)
    h = jnp.maximum(h + b1v[None, :], 0.0).astype(jnp.bfloat16)
    zero = jnp.zeros((1, BOT), dtype=jnp.bfloat16)
    hp = jnp.concatenate([zero, h, zero], axis=0)
    acc = jax.lax.dot_general(hp[0:L], w2[0], _DN_CONTRACT1,
                              preferred_element_type=jnp.float32)
    acc += jax.lax.dot_general(hp[1:L + 1], w2[1], _DN_CONTRACT1,
                               preferred_element_type=jnp.float32)
    acc += jax.lax.dot_general(hp[2:L + 2], w2[2], _DN_CONTRACT1,
                               preferred_element_type=jnp.float32)
    h2 = jnp.maximum(acc + b2v[None, :], 0.0).astype(jnp.bfloat16)
    y = jnp.dot(h2, w3, preferred_element_type=jnp.float32)
    return y + b3v[None, :]


def _dispatch_kernel(idx_ref, gv_ref, x_ref, w1_ref, w2_ref, w3_ref,
                     b1_ref, b2_ref, b3_ref, y_ref):
    i = pl.program_id(0)
    for di in range(G):
        b = i * G + di
        xb = x_ref[:, di, :]
        xb_bf = xb.astype(jnp.bfloat16)
        acc = None
        for k in range(K):
            e = idx_ref[b, k]
            g = gv_ref[b, k]
            y = _expert_chain(xb_bf, w1_ref[e], w2_ref[e], w3_ref[e],
                              b1_ref[e], b2_ref[e], b3_ref[e])
            term = g * jnp.maximum(y + xb, 0.0)
            acc = term if acc is None else acc + term
        y_ref[:, di, :] = acc


@jax.jit
def kernel(x, w_gate, w1, b1, w2, b2, w3, b3):
    xt = jnp.transpose(x, (2, 0, 1))
    wgt = jnp.transpose(w_gate, (1, 0))
    w2t = jnp.transpose(w2, (0, 3, 1, 2))
    w1f = w1.reshape(E, BOT, C)
    w3f = jnp.transpose(w3, (0, 2, 3, 1)).reshape(E, BOT, C)

    idx, gv, loss, w1b, w2b, w3b = pl.pallas_call(
        _gating_kernel,
        grid=(L // LCH,),
        in_specs=[
            pl.BlockSpec((LCH, B, C), lambda i: (i, 0, 0)),
            pl.BlockSpec((E, C), lambda i: (0, 0)),
            pl.BlockSpec((E, BOT, C), lambda i: (0, 0, 0)),
            pl.BlockSpec((E, 3, BOT, BOT), lambda i: (0, 0, 0, 0)),
            pl.BlockSpec((E, BOT, C), lambda i: (0, 0, 0)),
        ],
        out_specs=(
            pl.BlockSpec((B, K), lambda i: (0, 0)),
            pl.BlockSpec((B, K), lambda i: (0, 0)),
            pl.BlockSpec((1, 1), lambda i: (0, 0)),
            pl.BlockSpec((E, BOT, C), lambda i: (0, 0, 0)),
            pl.BlockSpec((E, 3, BOT, BOT), lambda i: (0, 0, 0, 0)),
            pl.BlockSpec((E, BOT, C), lambda i: (0, 0, 0)),
        ),
        out_shape=(
            jax.ShapeDtypeStruct((B, K), jnp.int32),
            jax.ShapeDtypeStruct((B, K), jnp.float32),
            jax.ShapeDtypeStruct((1, 1), jnp.float32),
            jax.ShapeDtypeStruct((E, BOT, C), jnp.bfloat16),
            jax.ShapeDtypeStruct((E, 3, BOT, BOT), jnp.bfloat16),
            jax.ShapeDtypeStruct((E, BOT, C), jnp.bfloat16),
        ),
        scratch_shapes=[pltpu.VMEM((B, C), jnp.float32)],
    )(xt, wgt, w1f, w2t, w3f)

    grid_spec = pltpu.PrefetchScalarGridSpec(
        num_scalar_prefetch=2,
        grid=(B // G,),
        in_specs=[
            pl.BlockSpec((L, G, C), lambda i, idx, gv: (0, i, 0)),
            pl.BlockSpec((E, BOT, C), lambda i, idx, gv: (0, 0, 0)),
            pl.BlockSpec((E, 3, BOT, BOT), lambda i, idx, gv: (0, 0, 0, 0)),
            pl.BlockSpec((E, BOT, C), lambda i, idx, gv: (0, 0, 0)),
            pl.BlockSpec((E, BOT), lambda i, idx, gv: (0, 0)),
            pl.BlockSpec((E, BOT), lambda i, idx, gv: (0, 0)),
            pl.BlockSpec((E, C), lambda i, idx, gv: (0, 0)),
        ],
        out_specs=pl.BlockSpec((L, G, C), lambda i, idx, gv: (0, i, 0)),
    )
    yt = pl.pallas_call(
        _dispatch_kernel,
        grid_spec=grid_spec,
        out_shape=jax.ShapeDtypeStruct((L, B, C), jnp.float32),
    )(idx, gv, xt, w1b, w2b, w3b, b1, b2, b3)

    y = jnp.transpose(yt, (1, 2, 0))
    return (y, loss.reshape(()))
